# Initial kernel scaffold; baseline (speedup 1.0000x reference)
import jax
import jax.numpy as jnp
from jax.experimental import pallas as pl


def kernel(x, W_down, b_down, W_up, b_up, alpha, W1, b1, g1, be1, W2, b2, g2, be2):
    raise NotImplementedError("write your pallas kernel here")


# speedup vs baseline: 1266.7299x; 1266.7299x over previous
import functools

import jax
import jax.numpy as jnp
from jax import lax
from jax.experimental import pallas as pl

N = 512


def _l2norm(x, eps=1e-12):
    n = jnp.sqrt(jnp.sum(x * x, axis=-1, keepdims=True))
    return x / jnp.maximum(n, eps)


def _layernorm(x, g, b, eps=1e-5):
    mu = jnp.mean(x, axis=-1, keepdims=True)
    var = jnp.var(x, axis=-1, keepdims=True)
    return (x - mu) / jnp.sqrt(var + eps) * g + b


def _body(x_ref, Wd_ref, bd_ref, Wu_ref, bu_ref, alpha_ref,
          W1_ref, b1_ref, g1_ref, be1_ref, W2_ref, b2_ref, g2_ref, be2_ref,
          adapted_ref, xg_ref):
    f32 = jnp.float32
    x = x_ref[...]
    alpha = alpha_ref[0, 0]

    h = jnp.maximum(
        lax.dot_general(x, Wd_ref[...], (((1,), (0,)), ((), ())),
                        preferred_element_type=f32) + bd_ref[...], 0.0)
    h = lax.dot_general(h, Wu_ref[...], (((1,), (0,)), ((), ())),
                        preferred_element_type=f32) + bu_ref[...]
    adapted = alpha * (h + x) + (1.0 - alpha) * x
    adapted = _l2norm(adapted)
    adapted_ref[...] = adapted

    fn = _l2norm(adapted)
    S = lax.dot_general(fn, fn, (((1,), (1,)), ((), ())),
                        preferred_element_type=f32)
    r = lax.broadcasted_iota(jnp.int32, (N, N), 0)
    c = lax.broadcasted_iota(jnp.int32, (N, N), 1)
    S = jnp.where(r == c, 0.0, S)
    S = jnp.maximum(S, 0.0) + 1e-6
    S = S / jnp.maximum(jnp.sum(S, axis=-1, keepdims=True), 1e-6)

    deg = jnp.sum(S, axis=0, keepdims=True)
    dinv = jnp.where(deg > 0.0, lax.rsqrt(deg), 0.0)
    A = S * dinv.reshape(N, 1) * dinv

    xw = lax.dot_general(adapted, W1_ref[...], (((1,), (0,)), ((), ())),
                         preferred_element_type=f32)
    out = lax.dot_general(A, xw, (((0,), (0,)), ((), ())),
                          preferred_element_type=f32) + b1_ref[...]
    out = _l2norm(jnp.maximum(_layernorm(out, g1_ref[...], be1_ref[...]), 0.0))

    xw = lax.dot_general(out, W2_ref[...], (((1,), (0,)), ((), ())),
                         preferred_element_type=f32)
    out = lax.dot_general(A, xw, (((0,), (0,)), ((), ())),
                          preferred_element_type=f32) + b2_ref[...]
    out = _l2norm(jnp.maximum(_layernorm(out, g2_ref[...], be2_ref[...]), 0.0))
    xg_ref[...] = out


@functools.partial(jax.jit, static_argnames=())
def kernel(x, W_down, b_down, W_up, b_up, alpha, W1, b1, g1, be1,
           W2, b2, g2, be2):
    out_shape = (
        jax.ShapeDtypeStruct((N, N), jnp.float32),
        jax.ShapeDtypeStruct((N, N), jnp.float32),
    )
    adapted, xg = pl.pallas_call(
        _body,
        out_shape=out_shape,
    )(
        x,
        W_down, b_down.reshape(1, -1),
        W_up, b_up.reshape(1, -1),
        jnp.reshape(alpha, (1, 1)),
        W1, b1.reshape(1, -1), g1.reshape(1, -1), be1.reshape(1, -1),
        W2, b2.reshape(1, -1), g2.reshape(1, -1), be2.reshape(1, -1),
    )
    return adapted, xg
